# baseline (speedup 1.0000x reference)
import functools
import jax
import jax.numpy as jnp
from jax import lax
from jax.experimental import pallas as pl
from jax.experimental.pallas import tpu as pltpu
from jax.experimental.pallas import tpu_sc as plsc

MAX_STEPS = 204.0
L = 16
HEAD = 16
TAIL = 16
U = 8


def _make_sc_shift(R, R_active, T, C):
    info = plsc.get_sparse_core_info()
    NC, NS = info.num_cores, info.num_subcores
    NW = NC * NS
    assert R_active % (2 * NW) == 0 and C % L == 0
    nchunks = T // L
    n_int = nchunks - HEAD - TAIL
    assert T % L == 0 and n_int % U == 0 and HEAD * L >= MAX_STEPS + 1
    rows_per = R_active // NW
    pairs = rows_per // 2

    mesh = plsc.VectorSubcoreMesh(core_axis_name="c", subcore_axis_name="s")

    @functools.partial(
        pl.kernel,
        mesh=mesh,
        out_type=jax.ShapeDtypeStruct((R_active, T), jnp.float32),
        compiler_params=pltpu.CompilerParams(needs_layout_passes=False),
        scratch_types=[
            pltpu.VMEM((C,), jnp.float32),
            pltpu.VMEM((C,), jnp.int32),
            pltpu.VMEM((C,), jnp.float32),
            pltpu.VMEM((T,), jnp.float32),
            pltpu.VMEM((T,), jnp.float32),
            pltpu.VMEM((T,), jnp.float32),
            pltpu.VMEM((T,), jnp.float32),
            pltpu.SemaphoreType.DMA,
            pltpu.SemaphoreType.DMA,
            pltpu.SemaphoreType.DMA,
            pltpu.SemaphoreType.DMA,
        ],
    )
    def sc_shift(x_hbm, shift_hbm, out_hbm, shiftv, kbuf, abuf,
                 in0, in1, ob0, ob1, si0, si1, so0, so1):
        wid = lax.axis_index("s") * NC + lax.axis_index("c")
        pltpu.sync_copy(shift_hbm, shiftv)

        for i in range(C // L):
            p = shiftv[pl.ds(i * L, L)]
            e = jnp.exp(-2.0 * jnp.abs(p))
            s = jnp.sign(p) * ((1.0 - e) / (1.0 + e)) * MAX_STEPS
            tr = s.astype(jnp.int32)
            kf = jnp.where(tr.astype(jnp.float32) > s, tr - 1, tr)
            kbuf[pl.ds(i * L, L)] = kf
            abuf[pl.ds(i * L, L)] = s - kf.astype(jnp.float32)

        iota = lax.iota(jnp.int32, L)
        base_row = wid * rows_per

        def row_params(row):
            ch = jnp.full((L,), lax.rem(row, C), jnp.int32)
            av = plsc.load_gather(abuf, [ch])
            base = plsc.load_gather(kbuf, [ch]) + iota
            return av, 1.0 - av, base

        def clip_chunk(inb, ob, av, bv, base, t):
            t = pl.multiple_of(t, L)
            idx = base + t
            i0 = jnp.minimum(jnp.maximum(idx, 0), T - 1)
            i1 = jnp.minimum(jnp.maximum(idx + 1, 0), T - 1)
            v0 = plsc.load_gather(inb, [i0])
            v1 = plsc.load_gather(inb, [i1])
            ob[pl.ds(t, L)] = bv * v0 + av * v1

        def compute_row(inb, ob, av, bv, base):
            @plsc.parallel_loop(0, HEAD, unroll=8)
            def _(ci):
                clip_chunk(inb, ob, av, bv, base, ci * L)

            @plsc.parallel_loop(HEAD, nchunks - TAIL, unroll=U)
            def _(ci):
                t = pl.multiple_of(ci * L, L)
                i0 = base + t
                v0 = plsc.load_gather(inb, [i0])
                v1 = plsc.load_gather(inb, [i0 + 1])
                ob[pl.ds(t, L)] = bv * v0 + av * v1

            @plsc.parallel_loop(nchunks - TAIL, nchunks, unroll=8)
            def _(ci):
                clip_chunk(inb, ob, av, bv, base, ci * L)

        pltpu.make_async_copy(x_hbm.at[base_row], in0, si0).start()

        def pair_body(p, _):
            r0 = base_row + 2 * p
            pltpu.make_async_copy(x_hbm.at[r0 + 1], in1, si1).start()
            pltpu.make_async_copy(x_hbm.at[r0], in0, si0).wait()
            av, bv, base = row_params(r0)

            @pl.when(p > 0)
            def _():
                pltpu.make_async_copy(ob0, out_hbm.at[r0], so0).wait()

            compute_row(in0, ob0, av, bv, base)
            pltpu.make_async_copy(ob0, out_hbm.at[r0], so0).start()

            @pl.when(p < pairs - 1)
            def _():
                pltpu.make_async_copy(x_hbm.at[r0 + 2], in0, si0).start()

            pltpu.make_async_copy(x_hbm.at[r0 + 1], in1, si1).wait()
            av1, bv1, base1 = row_params(r0 + 1)

            @pl.when(p > 0)
            def _():
                pltpu.make_async_copy(ob1, out_hbm.at[r0 + 1], so1).wait()

            compute_row(in1, ob1, av1, bv1, base1)
            pltpu.make_async_copy(ob1, out_hbm.at[r0 + 1], so1).start()
            return 0

        lax.fori_loop(0, pairs, pair_body, 0)
        pltpu.make_async_copy(ob0, out_hbm.at[base_row], so0).wait()
        pltpu.make_async_copy(ob1, out_hbm.at[base_row + 1], so1).wait()

    return sc_shift


def _make_tc_shift(B_all, B1, C, T, row_blk):
    def body(shift_ref, x_ref, o_ref):
        steps = jnp.tanh(shift_ref[0, 0, 0]) * MAX_STEPS
        kf = jnp.floor(steps)
        k = kf.astype(jnp.int32)
        a = steps - kf
        xb = x_ref[...]
        r0 = pltpu.roll(xb, lax.rem(T - k, T), axis=1)
        r1 = pltpu.roll(xb, lax.rem(2 * T - k - 1, T), axis=1)
        tpk = jax.lax.broadcasted_iota(jnp.int32, (B1, T), 1) + k
        x0 = xb[:, 0:1]
        xT = xb[:, T - 1:T]
        y0 = jnp.where(tpk < 0, x0, jnp.where(tpk > T - 1, xT, r0))
        y1 = jnp.where(tpk + 1 < 0, x0, jnp.where(tpk + 1 > T - 1, xT, r1))
        o_ref[...] = (1.0 - a) * y0 + a * y1

    return pl.pallas_call(
        body,
        grid=(C,),
        in_specs=[
            pl.BlockSpec((1, 1, 1), lambda c: (c, 0, 0), memory_space=pltpu.SMEM),
            pl.BlockSpec((B1, T), lambda c: (row_blk, c)),
        ],
        out_specs=pl.BlockSpec((B1, T), lambda c: (0, c)),
        out_shape=jax.ShapeDtypeStruct((B1, C * T), jnp.float32),
    )


B_SC = 16


def kernel(x, shift_param):
    B, C, T = x.shape
    xr = x.reshape(B * C, T)
    sp = shift_param.reshape(C).astype(jnp.float32)
    b0 = min(B_SC, B)
    sc_out = _make_sc_shift(B * C, b0 * C, T, C)(xr, sp)
    parts = [sc_out.reshape(b0, C, T)]
    if b0 < B:
        b1 = B - b0
        tc_out = _make_tc_shift(B, b1, C, T, b0 // b1)(
            sp.reshape(C, 1, 1), x.reshape(B, C * T))
        parts.append(tc_out.reshape(b1, C, T))
    return jnp.concatenate(parts, axis=0) if len(parts) > 1 else parts[0]


# speedup vs baseline: 3.8665x; 3.8665x over previous
import functools
import jax
import jax.numpy as jnp
from jax import lax
from jax.experimental import pallas as pl
from jax.experimental.pallas import tpu as pltpu
from jax.experimental.pallas import tpu_sc as plsc

MAX_STEPS = 204.0
L = 16
HEAD = 16
TAIL = 16
U = 8
Q = 4


def _make_sc_shift(R, T, C):
    info = plsc.get_sparse_core_info()
    NC, NS = info.num_cores, info.num_subcores
    NW = NC * NS
    assert R % (Q * NW) == 0 and C % L == 0
    nchunks = T // L
    n_int = nchunks - HEAD - TAIL
    assert T % L == 0 and n_int % U == 0 and HEAD * L >= MAX_STEPS + 1
    rows_per = R // NW
    quads = rows_per // Q

    mesh = plsc.VectorSubcoreMesh(core_axis_name="c", subcore_axis_name="s")

    @functools.partial(
        pl.kernel,
        mesh=mesh,
        out_type=jax.ShapeDtypeStruct((R, T), jnp.float32),
        compiler_params=pltpu.CompilerParams(needs_layout_passes=False),
        scratch_types=[
            pltpu.VMEM((C,), jnp.float32),
            pltpu.VMEM((C,), jnp.int32),
            pltpu.VMEM((C,), jnp.float32),
            [pltpu.VMEM((T,), jnp.float32) for _ in range(Q)],
            [pltpu.VMEM((T,), jnp.float32) for _ in range(2)],
            [pltpu.SemaphoreType.DMA for _ in range(Q)],
            [pltpu.SemaphoreType.DMA for _ in range(2)],
        ],
    )
    def sc_shift(x_hbm, shift_hbm, out_hbm, shiftv, kbuf, abuf, ins, obs,
                 sis, sos):
        wid = lax.axis_index("s") * NC + lax.axis_index("c")
        pltpu.sync_copy(shift_hbm, shiftv)

        for i in range(C // L):
            p = shiftv[pl.ds(i * L, L)]
            e = jnp.exp(-2.0 * jnp.abs(p))
            s = jnp.sign(p) * ((1.0 - e) / (1.0 + e)) * MAX_STEPS
            tr = s.astype(jnp.int32)
            kf = jnp.where(tr.astype(jnp.float32) > s, tr - 1, tr)
            kbuf[pl.ds(i * L, L)] = kf
            abuf[pl.ds(i * L, L)] = s - kf.astype(jnp.float32)

        iota = lax.iota(jnp.int32, L)
        base_row = wid * rows_per

        def row_params(row):
            ch = jnp.full((L,), lax.rem(row, C), jnp.int32)
            av = plsc.load_gather(abuf, [ch])
            base = plsc.load_gather(kbuf, [ch]) + iota
            return av, 1.0 - av, base

        def clip_chunk(inb, ob, av, bv, base, t):
            t = pl.multiple_of(t, L)
            idx = base + t
            i0 = jnp.minimum(jnp.maximum(idx, 0), T - 1)
            i1 = jnp.minimum(jnp.maximum(idx + 1, 0), T - 1)
            v0 = plsc.load_gather(inb, [i0])
            v1 = plsc.load_gather(inb, [i1])
            ob[pl.ds(t, L)] = bv * v0 + av * v1

        def compute_row(inb, ob, av, bv, base):
            @plsc.parallel_loop(0, HEAD, unroll=8)
            def _(ci):
                clip_chunk(inb, ob, av, bv, base, ci * L)

            @plsc.parallel_loop(HEAD, nchunks - TAIL, unroll=U)
            def _(ci):
                t = pl.multiple_of(ci * L, L)
                i0 = base + t
                v0 = plsc.load_gather(inb, [i0])
                v1 = plsc.load_gather(inb, [i0 + 1])
                ob[pl.ds(t, L)] = bv * v0 + av * v1

            @plsc.parallel_loop(nchunks - TAIL, nchunks, unroll=8)
            def _(ci):
                clip_chunk(inb, ob, av, bv, base, ci * L)

        for u in range(Q - 1):
            pltpu.make_async_copy(x_hbm.at[base_row + u], ins[u], sis[u]).start()

        def quad_body(q, _):
            r0 = base_row + Q * q
            j0 = Q * q
            for u in range(Q):
                r = r0 + u
                nxt = (u + Q - 1) % Q

                @pl.when(j0 + u + Q - 1 < rows_per)
                def _():
                    pltpu.make_async_copy(
                        x_hbm.at[r + Q - 1], ins[nxt], sis[nxt]).start()

                pltpu.make_async_copy(x_hbm.at[r], ins[u], sis[u]).wait()
                av, bv, base = row_params(r)

                @pl.when(j0 + u >= 2)
                def _():
                    pltpu.make_async_copy(
                        obs[u % 2], out_hbm.at[r], sos[u % 2]).wait()

                compute_row(ins[u], obs[u % 2], av, bv, base)
                pltpu.make_async_copy(
                    obs[u % 2], out_hbm.at[r], sos[u % 2]).start()
            return 0

        lax.fori_loop(0, quads, quad_body, 0)
        pltpu.make_async_copy(obs[0], out_hbm.at[base_row], sos[0]).wait()
        pltpu.make_async_copy(obs[1], out_hbm.at[base_row + 1], sos[1]).wait()

    return sc_shift


def kernel(x, shift_param):
    B, C, T = x.shape
    xr = x.reshape(B * C, T)
    sp = shift_param.reshape(C).astype(jnp.float32)
    out = _make_sc_shift(B * C, T, C)(xr, sp)
    return out.reshape(B, C, T)


# speedup vs baseline: 3.8869x; 1.0053x over previous
import functools
import jax
import jax.numpy as jnp
from jax import lax
from jax.experimental import pallas as pl
from jax.experimental.pallas import tpu as pltpu
from jax.experimental.pallas import tpu_sc as plsc

MAX_STEPS = 204.0
L = 16
HEAD = 16
TAIL = 16
U = 8
Q = 4


def _make_sc_shift(R, T, C):
    info = plsc.get_sparse_core_info()
    NC, NS = info.num_cores, info.num_subcores
    NW = NC * NS
    assert R % (Q * NW) == 0 and C % L == 0
    nchunks = T // L
    n_int = nchunks - HEAD - TAIL
    assert T % L == 0 and n_int % U == 0 and HEAD * L >= MAX_STEPS + 1
    rows_per = R // NW
    quads = rows_per // Q

    mesh = plsc.VectorSubcoreMesh(core_axis_name="c", subcore_axis_name="s")

    @functools.partial(
        pl.kernel,
        mesh=mesh,
        out_type=jax.ShapeDtypeStruct((R, T), jnp.float32),
        compiler_params=pltpu.CompilerParams(needs_layout_passes=False),
        scratch_types=[
            pltpu.VMEM((C,), jnp.float32),
            pltpu.VMEM((C,), jnp.int32),
            pltpu.VMEM((C,), jnp.float32),
            [pltpu.VMEM((T,), jnp.float32) for _ in range(Q)],
            [pltpu.VMEM((T,), jnp.float32) for _ in range(Q)],
            [pltpu.SemaphoreType.DMA for _ in range(Q)],
            [pltpu.SemaphoreType.DMA for _ in range(Q)],
        ],
    )
    def sc_shift(x_hbm, shift_hbm, out_hbm, shiftv, kbuf, abuf, ins, obs,
                 sis, sos):
        wid = lax.axis_index("s") * NC + lax.axis_index("c")
        pltpu.sync_copy(shift_hbm, shiftv)

        for i in range(C // L):
            p = shiftv[pl.ds(i * L, L)]
            e = jnp.exp(-2.0 * jnp.abs(p))
            s = jnp.sign(p) * ((1.0 - e) / (1.0 + e)) * MAX_STEPS
            tr = s.astype(jnp.int32)
            kf = jnp.where(tr.astype(jnp.float32) > s, tr - 1, tr)
            kbuf[pl.ds(i * L, L)] = kf
            abuf[pl.ds(i * L, L)] = s - kf.astype(jnp.float32)

        iota = lax.iota(jnp.int32, L)
        base_row = wid * rows_per

        def row_params(row):
            ch = jnp.full((L,), lax.rem(row, C), jnp.int32)
            av = plsc.load_gather(abuf, [ch])
            base = plsc.load_gather(kbuf, [ch]) + iota
            return av, 1.0 - av, base

        def clip_chunk(inb, ob, av, bv, base, t):
            t = pl.multiple_of(t, L)
            idx = base + t
            i0 = jnp.minimum(jnp.maximum(idx, 0), T - 1)
            i1 = jnp.minimum(jnp.maximum(idx + 1, 0), T - 1)
            v0 = plsc.load_gather(inb, [i0])
            v1 = plsc.load_gather(inb, [i1])
            ob[pl.ds(t, L)] = bv * v0 + av * v1

        def compute_row(inb, ob, av, bv, base):
            @plsc.parallel_loop(0, HEAD, unroll=8)
            def _(ci):
                clip_chunk(inb, ob, av, bv, base, ci * L)

            @plsc.parallel_loop(HEAD, nchunks - TAIL, unroll=U)
            def _(ci):
                t = pl.multiple_of(ci * L, L)
                i0 = base + t
                v0 = plsc.load_gather(inb, [i0])
                v1 = plsc.load_gather(inb, [i0 + 1])
                ob[pl.ds(t, L)] = bv * v0 + av * v1

            @plsc.parallel_loop(nchunks - TAIL, nchunks, unroll=8)
            def _(ci):
                clip_chunk(inb, ob, av, bv, base, ci * L)

        for u in range(Q - 1):
            pltpu.make_async_copy(x_hbm.at[base_row + u], ins[u], sis[u]).start()

        def quad_body(q, _):
            r0 = base_row + Q * q
            j0 = Q * q
            for u in range(Q):
                r = r0 + u
                nxt = (u + Q - 1) % Q

                @pl.when(j0 + u + Q - 1 < rows_per)
                def _():
                    pltpu.make_async_copy(
                        x_hbm.at[r + Q - 1], ins[nxt], sis[nxt]).start()

                pltpu.make_async_copy(x_hbm.at[r], ins[u], sis[u]).wait()
                av, bv, base = row_params(r)

                @pl.when(j0 + u >= Q)
                def _():
                    pltpu.make_async_copy(
                        obs[u], out_hbm.at[r], sos[u]).wait()

                compute_row(ins[u], obs[u], av, bv, base)
                pltpu.make_async_copy(
                    obs[u], out_hbm.at[r], sos[u]).start()
            return 0

        lax.fori_loop(0, quads, quad_body, 0)
        for u in range(Q):
            pltpu.make_async_copy(
                obs[u], out_hbm.at[base_row + u], sos[u]).wait()

    return sc_shift


def kernel(x, shift_param):
    B, C, T = x.shape
    xr = x.reshape(B * C, T)
    sp = shift_param.reshape(C).astype(jnp.float32)
    out = _make_sc_shift(B * C, T, C)(xr, sp)
    return out.reshape(B, C, T)
